# baseline (speedup 1.0000x reference)
import functools

import jax
import jax.numpy as jnp
from jax import lax
from jax.experimental import pallas as pl
from jax.experimental.pallas import tpu as pltpu
from jax.experimental.pallas import tpu_sc as plsc

_BATCH = 16384
_DIM = 64
_NC = 2
_NS = 16
_NW = _NC * _NS
_BPW = _BATCH // _NW
_CHUNK = 128
_NCH = _BPW // _CHUNK
_NG = _CHUNK // 16


def _tri_vec_body(hid, rid, tid, e1, e2, e3, q1, q2, q3, out,
                  ih, ir, it,
                  bh1, bh2, bh3, br1, br2, br3, bt1, bt2, bt3,
                  score, sem):
    wid = lax.axis_index("s") * _NC + lax.axis_index("c")
    base = wid * _BPW
    pltpu.sync_copy(hid.at[pl.ds(base, _BPW)], ih)
    pltpu.sync_copy(rid.at[pl.ds(base, _BPW)], ir)
    pltpu.sync_copy(tid.at[pl.ds(base, _BPW)], it)

    lane = lax.broadcasted_iota(jnp.int32, (16,), 0)
    lane0 = lane == 0

    for c in range(_NCH):
        ihs = ih.at[pl.ds(c * _CHUNK, _CHUNK)]
        irs = ir.at[pl.ds(c * _CHUNK, _CHUNK)]
        its = it.at[pl.ds(c * _CHUNK, _CHUNK)]
        cps = [
            pltpu.async_copy(e1.at[ihs], bh1, sem),
            pltpu.async_copy(e2.at[ihs], bh2, sem),
            pltpu.async_copy(e3.at[ihs], bh3, sem),
            pltpu.async_copy(q1.at[irs], br1, sem),
            pltpu.async_copy(q2.at[irs], br2, sem),
            pltpu.async_copy(q3.at[irs], br3, sem),
            pltpu.async_copy(e1.at[its], bt1, sem),
            pltpu.async_copy(e2.at[its], bt2, sem),
            pltpu.async_copy(e3.at[its], bt3, sem),
        ]
        for cp in cps:
            cp.wait()

        def r_body(r, carry):
            acc = jnp.zeros((16,), jnp.float32)
            for k in range(_DIM // 16):
                sl = pl.ds(k * 16, 16)
                acc = (acc
                       + bh1[r, sl] * br1[r, sl] * bt3[r, sl]
                       + bh2[r, sl] * br2[r, sl] * bt2[r, sl]
                       + bh3[r, sl] * br3[r, sl] * bt1[r, sl])
            pos = jnp.full((16,), c * _CHUNK + r, jnp.int32)
            val = jnp.full((16,), jnp.sum(acc), jnp.float32)
            plsc.store_scatter(score, [pos], val, mask=lane0)
            return carry

        lax.fori_loop(0, _CHUNK, r_body, 0)

    pltpu.sync_copy(score, out.at[pl.ds(base, _BPW)])


@jax.jit
def _tri_vec(hid, rid, tid, e1, e2, e3, q1, q2, q3):
    fn = pl.kernel(
        _tri_vec_body,
        out_type=jax.ShapeDtypeStruct((_BATCH,), jnp.float32),
        mesh=plsc.VectorSubcoreMesh(core_axis_name="c", subcore_axis_name="s"),
        compiler_params=pltpu.CompilerParams(
            needs_layout_passes=False, use_tc_tiling_on_sc=False),
        scratch_types=[
            pltpu.VMEM((_BPW,), jnp.int32),
            pltpu.VMEM((_BPW,), jnp.int32),
            pltpu.VMEM((_BPW,), jnp.int32),
        ] + [pltpu.VMEM((_CHUNK, _DIM), jnp.float32)] * 9 + [
            pltpu.VMEM((_BPW,), jnp.float32),
            pltpu.SemaphoreType.DMA,
        ],
    )
    return fn(hid, rid, tid, e1, e2, e3, q1, q2, q3)


def kernel(data, ent_1, ent_2, ent_3, rel_1, rel_2, rel_3):
    h_idx = data[:, 0]
    r_idx = data[:, 1]
    t_idx = data[:, 2]
    return _tri_vec(h_idx, r_idx, t_idx, ent_1, ent_2, ent_3,
                    rel_1, rel_2, rel_3)


# speedup vs baseline: 1.2348x; 1.2348x over previous
import jax
import jax.numpy as jnp
from jax import lax
from jax.experimental import pallas as pl
from jax.experimental.pallas import tpu as pltpu
from jax.experimental.pallas import tpu_sc as plsc

_BATCH = 16384
_DIM = 64
_NC = 2
_NS = 16
_NW = _NC * _NS
_BPW = _BATCH // _NW
_G = 16
_NG = _BPW // _G


def _tri_vec_body(hid, rid, tid, e1, e2, e3, q1, q2, q3, out,
                  ih, ir, it, bufs0, bufs1, score, sem0, sem1):
    wid = lax.axis_index("s") * _NC + lax.axis_index("c")
    base = wid * _BPW
    pltpu.sync_copy(hid.at[pl.ds(base, _BPW)], ih)
    pltpu.sync_copy(rid.at[pl.ds(base, _BPW)], ir)
    pltpu.sync_copy(tid.at[pl.ds(base, _BPW)], it)

    lane = lax.broadcasted_iota(jnp.int32, (16,), 0)
    lane0 = lane == 0
    tables = (e1, e2, e3, q1, q2, q3, e1, e2, e3)

    def fire(g, bufs, sem):
        hv = ih[pl.ds(g * _G, _G)]
        rv = ir[pl.ds(g * _G, _G)]
        tv = it[pl.ds(g * _G, _G)]
        for j in range(_G):
            rows = (hv[j], hv[j], hv[j], rv[j], rv[j], rv[j],
                    tv[j], tv[j], tv[j])
            for i in range(9):
                pltpu.async_copy(tables[i].at[rows[i]], bufs.at[i, j], sem)

    def drain(bufs, sem):
        for i in range(9):
            for j in range(_G):
                pltpu.make_async_copy(e1.at[0], bufs.at[i, j], sem).wait()

    def compute(g, bufs):
        def r_body(r, carry):
            acc = jnp.zeros((16,), jnp.float32)
            for k in range(_DIM // 16):
                sl = pl.ds(k * 16, 16)
                acc = (acc
                       + bufs[0, r, sl] * bufs[3, r, sl] * bufs[8, r, sl]
                       + bufs[1, r, sl] * bufs[4, r, sl] * bufs[7, r, sl]
                       + bufs[2, r, sl] * bufs[5, r, sl] * bufs[6, r, sl])
            pos = g * _G + r + jnp.zeros((16,), jnp.int32)
            val = jnp.full((16,), jnp.sum(acc), jnp.float32)
            plsc.store_scatter(score, [pos], val, mask=lane0)
            return carry

        lax.fori_loop(0, _G, r_body, 0)

    fire(0, bufs0, sem0)

    def pair_body(p, carry):
        g0 = p * 2
        fire(g0 + 1, bufs1, sem1)
        drain(bufs0, sem0)
        compute(g0, bufs0)

        @pl.when(p < _NG // 2 - 1)
        def _():
            fire(g0 + 2, bufs0, sem0)

        drain(bufs1, sem1)
        compute(g0 + 1, bufs1)
        return carry

    lax.fori_loop(0, _NG // 2, pair_body, 0)

    pltpu.sync_copy(score, out.at[pl.ds(base, _BPW)])


@jax.jit
def _tri_vec(hid, rid, tid, e1, e2, e3, q1, q2, q3):
    fn = pl.kernel(
        _tri_vec_body,
        out_type=jax.ShapeDtypeStruct((_BATCH,), jnp.float32),
        mesh=plsc.VectorSubcoreMesh(core_axis_name="c", subcore_axis_name="s"),
        compiler_params=pltpu.CompilerParams(needs_layout_passes=False),
        scratch_types=[
            pltpu.VMEM((_BPW,), jnp.int32),
            pltpu.VMEM((_BPW,), jnp.int32),
            pltpu.VMEM((_BPW,), jnp.int32),
            pltpu.VMEM((9, _G, _DIM), jnp.float32),
            pltpu.VMEM((9, _G, _DIM), jnp.float32),
            pltpu.VMEM((_BPW,), jnp.float32),
            pltpu.SemaphoreType.DMA,
            pltpu.SemaphoreType.DMA,
        ],
    )
    return fn(hid, rid, tid, e1, e2, e3, q1, q2, q3)


def kernel(data, ent_1, ent_2, ent_3, rel_1, rel_2, rel_3):
    h_idx = data[:, 0]
    r_idx = data[:, 1]
    t_idx = data[:, 2]
    return _tri_vec(h_idx, r_idx, t_idx, ent_1, ent_2, ent_3,
                    rel_1, rel_2, rel_3)


# speedup vs baseline: 2.3988x; 1.9427x over previous
import jax
import jax.numpy as jnp
from jax import lax
from jax.experimental import pallas as pl
from jax.experimental.pallas import tpu as pltpu
from jax.experimental.pallas import tpu_sc as plsc

_BATCH = 16384
_DIM = 64
_ENT = 100000
_NC = 2
_NS = 16
_BPW = _BATCH // _NS
_CPC = _DIM // _NC
_GCH = 128
_NGC = _BPW // _GCH


def _tri_vec_body(hid, rid, tid, e1, e2, e3, q1, q2, q3, out,
                  ih, ir, it, gb, acc,
                  s00, s01, s02, s03, s04, s05,
                  s10, s11, s12, s13, s14, s15,
                  semg, semst0, semst1):
    core = lax.axis_index("c")
    sid = lax.axis_index("s")
    base = sid * _BPW
    pltpu.sync_copy(hid.at[pl.ds(base, _BPW)], ih)
    pltpu.sync_copy(rid.at[pl.ds(base, _BPW)], ir)
    pltpu.sync_copy(tid.at[pl.ds(base, _BPW)], it)

    tabs = (e1, e2, e3, q1, q2, q3)
    stage = ((s00, s01, s02, s03, s04, s05),
             (s10, s11, s12, s13, s14, s15))
    stsem = (semst0, semst1)
    roles = ((0, ih), (1, ih), (2, ih), (3, ir), (4, ir), (5, ir),
             (0, it), (1, it), (2, it))

    col0 = core * _CPC

    def fire_stage(col, p):
        for i in range(6):
            @pl.when(sid == i)
            def _():
                pltpu.async_copy(tabs[i].at[pl.ds(col, 1), :],
                                 stage[p][i], stsem[p])

    def drain_stage(p):
        for i in range(6):
            @pl.when(sid == i)
            def _():
                pltpu.make_async_copy(
                    tabs[i].at[pl.ds(0, 1), :], stage[p][i], stsem[p]).wait()

    def zero_acc(g, carry):
        acc[pl.ds(g * 16, 16)] = jnp.zeros((16,), jnp.float32)
        return carry

    lax.fori_loop(0, _BPW // 16, zero_acc, 0)

    fire_stage(col0, 0)
    fire_stage(col0 + 1, 1)

    def phase(col, p, t):
        drain_stage(p)
        plsc.subcore_barrier()
        cps = []
        for i, (tb, idx) in enumerate(roles):
            for k in range(_NGC):
                cps.append(pltpu.async_copy(
                    stage[p][tb].at[0].at[idx.at[pl.ds(k * _GCH, _GCH)]],
                    gb.at[pl.ds(i * _BPW + k * _GCH, _GCH)], semg))
        for cp in cps:
            cp.wait()

        def g_body(g, carry):
            o = g * 16
            def gs(i):
                return gb[pl.ds(i * _BPW + o, 16)]
            sl = pl.ds(o, 16)
            acc[sl] = (acc[sl]
                       + gs(0) * gs(3) * gs(8)
                       + gs(1) * gs(4) * gs(7)
                       + gs(2) * gs(5) * gs(6))
            return carry

        lax.fori_loop(0, _BPW // 16, g_body, 0)
        plsc.subcore_barrier()

        @pl.when(t < _CPC // 2 - 1)
        def _():
            fire_stage(col + 2, p)

    def pair_body(t, carry):
        col = col0 + 2 * t
        phase(col, 0, t)
        phase(col + 1, 1, t)
        return carry

    lax.fori_loop(0, _CPC // 2, pair_body, 0)

    pltpu.sync_copy(acc, out.at[pl.ds(core * _BATCH + base, _BPW)])


@jax.jit
def _tri_vec(hid, rid, tid, e1, e2, e3, q1, q2, q3):
    fn = pl.kernel(
        _tri_vec_body,
        out_type=jax.ShapeDtypeStruct((_NC * _BATCH,), jnp.float32),
        mesh=plsc.VectorSubcoreMesh(core_axis_name="c", subcore_axis_name="s"),
        compiler_params=pltpu.CompilerParams(needs_layout_passes=False),
        scratch_types=[
            pltpu.VMEM((_BPW,), jnp.int32),
            pltpu.VMEM((_BPW,), jnp.int32),
            pltpu.VMEM((_BPW,), jnp.int32),
            pltpu.VMEM((9 * _BPW,), jnp.float32),
            pltpu.VMEM((_BPW,), jnp.float32),
        ] + [pltpu.VMEM_SHARED((1, _ENT), jnp.float32)] * 12 + [
            pltpu.SemaphoreType.DMA,
            pltpu.SemaphoreType.DMA,
            pltpu.SemaphoreType.DMA,
        ],
    )
    return fn(hid, rid, tid, e1, e2, e3, q1, q2, q3)


def kernel(data, ent_1, ent_2, ent_3, rel_1, rel_2, rel_3):
    h_idx = data[:, 0]
    r_idx = data[:, 1]
    t_idx = data[:, 2]
    part = _tri_vec(h_idx, r_idx, t_idx, ent_1.T, ent_2.T, ent_3.T,
                    rel_1.T, rel_2.T, rel_3.T)
    return part[:_BATCH] + part[_BATCH:]


# speedup vs baseline: 2.4073x; 1.0035x over previous
import jax
import jax.numpy as jnp
from jax import lax
from jax.experimental import pallas as pl
from jax.experimental.pallas import tpu as pltpu
from jax.experimental.pallas import tpu_sc as plsc

_BATCH = 16384
_DIM = 64
_ENT = 100000
_NC = 2
_NS = 16
_BPW = _BATCH // _NS
_CPC = _DIM // _NC
_GCH = 128
_NGC = _BPW // _GCH


def _tri_vec_body(hid, rid, tid, e1, e2, e3, q1, q2, q3, out,
                  ih, ir, it, gb, acc,
                  s00, s01, s02, s03, s04, s05,
                  s10, s11, s12, s13, s14, s15,
                  semg, semst0, semst1):
    core = lax.axis_index("c")
    sid = lax.axis_index("s")
    base = sid * _BPW
    pltpu.sync_copy(hid.at[pl.ds(base, _BPW)], ih)
    pltpu.sync_copy(rid.at[pl.ds(base, _BPW)], ir)
    pltpu.sync_copy(tid.at[pl.ds(base, _BPW)], it)

    tabs = (e1, e2, e3, q1, q2, q3)
    stage = ((s00, s01, s02, s03, s04, s05),
             (s10, s11, s12, s13, s14, s15))
    stsem = (semst0, semst1)
    roles = ((0, ih), (1, ih), (2, ih), (3, ir), (4, ir), (5, ir),
             (0, it), (1, it), (2, it))

    col0 = core * _CPC

    def fire_stage(col, p):
        for i in range(6):
            @pl.when(sid == i)
            def _():
                pltpu.async_copy(tabs[i].at[pl.ds(col, 1), :],
                                 stage[p][i], stsem[p])

    def drain_stage(p):
        for i in range(6):
            @pl.when(sid == i)
            def _():
                pltpu.make_async_copy(
                    tabs[i].at[pl.ds(0, 1), :], stage[p][i], stsem[p]).wait()

    def zero_acc(g, carry):
        acc[pl.ds(g * 16, 16)] = jnp.zeros((16,), jnp.float32)
        return carry

    lax.fori_loop(0, _BPW // 16, zero_acc, 0)

    fire_stage(col0, 0)
    fire_stage(col0 + 1, 1)

    def phase(col, p, t):
        drain_stage(p)
        plsc.subcore_barrier()
        cps = []
        for i, (tb, idx) in enumerate(roles):
            for k in range(_NGC):
                cps.append(pltpu.async_copy(
                    stage[p][tb].at[0].at[idx.at[pl.ds(k * _GCH, _GCH)]],
                    gb.at[pl.ds(i * _BPW + k * _GCH, _GCH)], semg))
        for cp in cps:
            cp.wait()
        plsc.subcore_barrier()

        @pl.when(t < _CPC // 2 - 1)
        def _():
            fire_stage(col + 2, p)

        def g_body(g, carry):
            o = g * 16
            def gs(i):
                return gb[pl.ds(i * _BPW + o, 16)]
            sl = pl.ds(o, 16)
            acc[sl] = (acc[sl]
                       + gs(0) * gs(3) * gs(8)
                       + gs(1) * gs(4) * gs(7)
                       + gs(2) * gs(5) * gs(6))
            return carry

        lax.fori_loop(0, _BPW // 16, g_body, 0)

    def pair_body(t, carry):
        col = col0 + 2 * t
        phase(col, 0, t)
        phase(col + 1, 1, t)
        return carry

    lax.fori_loop(0, _CPC // 2, pair_body, 0)

    pltpu.sync_copy(acc, out.at[pl.ds(core * _BATCH + base, _BPW)])


@jax.jit
def _tri_vec(hid, rid, tid, e1, e2, e3, q1, q2, q3):
    fn = pl.kernel(
        _tri_vec_body,
        out_type=jax.ShapeDtypeStruct((_NC * _BATCH,), jnp.float32),
        mesh=plsc.VectorSubcoreMesh(core_axis_name="c", subcore_axis_name="s"),
        compiler_params=pltpu.CompilerParams(needs_layout_passes=False),
        scratch_types=[
            pltpu.VMEM((_BPW,), jnp.int32),
            pltpu.VMEM((_BPW,), jnp.int32),
            pltpu.VMEM((_BPW,), jnp.int32),
            pltpu.VMEM((9 * _BPW,), jnp.float32),
            pltpu.VMEM((_BPW,), jnp.float32),
        ] + [pltpu.VMEM_SHARED((1, _ENT), jnp.float32)] * 12 + [
            pltpu.SemaphoreType.DMA,
            pltpu.SemaphoreType.DMA,
            pltpu.SemaphoreType.DMA,
        ],
    )
    return fn(hid, rid, tid, e1, e2, e3, q1, q2, q3)


def kernel(data, ent_1, ent_2, ent_3, rel_1, rel_2, rel_3):
    h_idx = data[:, 0]
    r_idx = data[:, 1]
    t_idx = data[:, 2]
    part = _tri_vec(h_idx, r_idx, t_idx, ent_1.T, ent_2.T, ent_3.T,
                    rel_1.T, rel_2.T, rel_3.T)
    return part[:_BATCH] + part[_BATCH:]


# speedup vs baseline: 2.6837x; 1.1148x over previous
import jax
import jax.numpy as jnp
from jax import lax
from jax.experimental import pallas as pl
from jax.experimental.pallas import tpu as pltpu
from jax.experimental.pallas import tpu_sc as plsc

_BATCH = 16384
_DIM = 64
_ENT = 100000
_NC = 2
_NS = 16
_BPW = _BATCH // _NS
_CPC = _DIM // _NC
_GCH = 128
_NGC = _BPW // _GCH


def _tri_vec_body(hid, rid, tid, e1, e2, e3, q1, q2, q3, out,
                  ih, ir, it, gb, gc, acc,
                  s00, s01, s02, s03, s04, s05,
                  s10, s11, s12, s13, s14, s15,
                  semg, semh, semst0, semst1):
    core = lax.axis_index("c")
    sid = lax.axis_index("s")
    base = sid * _BPW
    pltpu.sync_copy(hid.at[pl.ds(base, _BPW)], ih)
    pltpu.sync_copy(rid.at[pl.ds(base, _BPW)], ir)
    pltpu.sync_copy(tid.at[pl.ds(base, _BPW)], it)

    tabs = (e1, e2, e3, q1, q2, q3)
    stage = ((s00, s01, s02, s03, s04, s05),
             (s10, s11, s12, s13, s14, s15))
    stsem = (semst0, semst1)
    roles = ((0, ih), (1, ih), (2, ih), (3, ir), (4, ir), (5, ir),
             (0, it), (1, it), (2, it))

    col0 = core * _CPC

    def fire_stage(col, p):
        for i in range(6):
            @pl.when(sid == i)
            def _():
                pltpu.async_copy(tabs[i].at[pl.ds(col, 1), :],
                                 stage[p][i], stsem[p])

    def drain_stage(p):
        for i in range(6):
            @pl.when(sid == i)
            def _():
                pltpu.make_async_copy(
                    tabs[i].at[pl.ds(0, 1), :], stage[p][i], stsem[p]).wait()

    def zero_acc(g, carry):
        acc[pl.ds(g * 16, 16)] = jnp.zeros((16,), jnp.float32)
        return carry

    lax.fori_loop(0, _BPW // 16, zero_acc, 0)

    gbs = (gb, gc)
    semgs = (semg, semh)

    def fire_gathers(p):
        for i, (tb, idx) in enumerate(roles):
            for k in range(_NGC):
                pltpu.async_copy(
                    stage[p][tb].at[0].at[idx.at[pl.ds(k * _GCH, _GCH)]],
                    gbs[p].at[pl.ds(i * _BPW + k * _GCH, _GCH)], semgs[p])

    def drain_gathers(p):
        for i, (tb, idx) in enumerate(roles):
            for k in range(_NGC):
                pltpu.make_async_copy(
                    stage[p][tb].at[0].at[idx.at[pl.ds(k * _GCH, _GCH)]],
                    gbs[p].at[pl.ds(i * _BPW + k * _GCH, _GCH)],
                    semgs[p]).wait()

    fire_stage(col0, 0)
    fire_stage(col0 + 1, 1)
    drain_stage(0)
    plsc.subcore_barrier()
    fire_gathers(0)

    def phase(col, p, t, ph):
        drain_gathers(p)
        plsc.subcore_barrier()

        @pl.when(t < _CPC // 2 - 1)
        def _():
            fire_stage(col + 2, p)

        q = 1 - p
        if ph == 0:
            drain_stage(q)
            plsc.subcore_barrier()
            fire_gathers(q)
        else:
            @pl.when(t < _CPC // 2 - 1)
            def _():
                drain_stage(q)
                plsc.subcore_barrier()
                fire_gathers(q)

        def g_body(g, carry):
            o = g * 16
            def gs(i):
                return gbs[p][pl.ds(i * _BPW + o, 16)]
            sl = pl.ds(o, 16)
            acc[sl] = (acc[sl]
                       + gs(0) * gs(3) * gs(8)
                       + gs(1) * gs(4) * gs(7)
                       + gs(2) * gs(5) * gs(6))
            return carry

        lax.fori_loop(0, _BPW // 16, g_body, 0)

    def pair_body(t, carry):
        col = col0 + 2 * t
        phase(col, 0, t, 0)
        phase(col + 1, 1, t, 1)
        return carry

    lax.fori_loop(0, _CPC // 2, pair_body, 0)

    pltpu.sync_copy(acc, out.at[pl.ds(core * _BATCH + base, _BPW)])


@jax.jit
def _tri_vec(hid, rid, tid, e1, e2, e3, q1, q2, q3):
    fn = pl.kernel(
        _tri_vec_body,
        out_type=jax.ShapeDtypeStruct((_NC * _BATCH,), jnp.float32),
        mesh=plsc.VectorSubcoreMesh(core_axis_name="c", subcore_axis_name="s"),
        compiler_params=pltpu.CompilerParams(needs_layout_passes=False),
        scratch_types=[
            pltpu.VMEM((_BPW,), jnp.int32),
            pltpu.VMEM((_BPW,), jnp.int32),
            pltpu.VMEM((_BPW,), jnp.int32),
            pltpu.VMEM((9 * _BPW,), jnp.float32),
            pltpu.VMEM((9 * _BPW,), jnp.float32),
            pltpu.VMEM((_BPW,), jnp.float32),
        ] + [pltpu.VMEM_SHARED((1, _ENT), jnp.float32)] * 12 + [
            pltpu.SemaphoreType.DMA,
            pltpu.SemaphoreType.DMA,
            pltpu.SemaphoreType.DMA,
            pltpu.SemaphoreType.DMA,
        ],
    )
    return fn(hid, rid, tid, e1, e2, e3, q1, q2, q3)


def kernel(data, ent_1, ent_2, ent_3, rel_1, rel_2, rel_3):
    h_idx = data[:, 0]
    r_idx = data[:, 1]
    t_idx = data[:, 2]
    part = _tri_vec(h_idx, r_idx, t_idx, ent_1.T, ent_2.T, ent_3.T,
                    rel_1.T, rel_2.T, rel_3.T)
    return part[:_BATCH] + part[_BATCH:]
